# baseline (speedup 1.0000x reference)
import functools

import jax
import jax.numpy as jnp
from jax import lax
from jax.experimental import pallas as pl
from jax.experimental.pallas import tpu as pltpu
from jax.experimental.pallas import tpu_sc as plsc

B = 32
N = 8192
G = 128
K = 32
EMBED_DIM = 768
HIDDEN = 512

NC = 2
NS = 16


def _fps_body(x_ref, y_ref, z_ref, idx_ref, cx_ref, cy_ref, cz_ref):
    x = x_ref[:]
    y = y_ref[:]
    z = z_ref[:]
    f32 = jnp.float32
    mx = jnp.mean(x, axis=1, keepdims=True)
    my = jnp.mean(y, axis=1, keepdims=True)
    mz = jnp.mean(z, axis=1, keepdims=True)
    dx, dy, dz = x - mx, y - my, z - mz
    dist0 = dx * dx + dy * dy + dz * dz
    iota = lax.broadcasted_iota(jnp.int32, (B, N), 1)

    def argmax_first(d):
        m = jnp.max(d, axis=1, keepdims=True)
        return jnp.min(jnp.where(d == m, iota, N), axis=1, keepdims=True)

    farthest = argmax_first(dist0)
    distance = jnp.full((B, N), 1e10, dtype=f32)
    iota_g = lax.broadcasted_iota(jnp.int32, (B, G), 1)
    acc_i = jnp.zeros((B, G), jnp.int32)
    acc_x = jnp.zeros((B, G), f32)
    acc_y = jnp.zeros((B, G), f32)
    acc_z = jnp.zeros((B, G), f32)

    def step(i, carry):
        distance, farthest, acc_i, acc_x, acc_y, acc_z = carry
        sel = iota == farthest
        zero = jnp.zeros((), f32)
        cx = jnp.sum(jnp.where(sel, x, zero), axis=1, keepdims=True)
        cy = jnp.sum(jnp.where(sel, y, zero), axis=1, keepdims=True)
        cz = jnp.sum(jnp.where(sel, z, zero), axis=1, keepdims=True)
        here = iota_g == i
        acc_i = jnp.where(here, farthest, acc_i)
        acc_x = jnp.where(here, cx, acc_x)
        acc_y = jnp.where(here, cy, acc_y)
        acc_z = jnp.where(here, cz, acc_z)
        ex, ey, ez = x - cx, y - cy, z - cz
        dist = ex * ex + ey * ey + ez * ez
        distance = jnp.minimum(distance, dist)
        farthest = argmax_first(distance)
        return distance, farthest, acc_i, acc_x, acc_y, acc_z

    carry = (distance, farthest, acc_i, acc_x, acc_y, acc_z)
    carry = lax.fori_loop(0, G, step, carry)
    _, _, acc_i, acc_x, acc_y, acc_z = carry
    idx_ref[:] = acc_i
    cx_ref[:] = acc_x
    cy_ref[:] = acc_y
    cz_ref[:] = acc_z


def _fps_pallas(xp, yp, zp):
    out_shapes = (
        jax.ShapeDtypeStruct((B, G), jnp.int32),
        jax.ShapeDtypeStruct((B, G), jnp.float32),
        jax.ShapeDtypeStruct((B, G), jnp.float32),
        jax.ShapeDtypeStruct((B, G), jnp.float32),
    )
    return pl.pallas_call(_fps_body, out_shape=out_shapes)(xp, yp, zp)


_NCHUNK = 64
_CW = N // _NCHUNK


def _dist_body(x_ref, y_ref, z_ref, cx_ref, cy_ref, cz_ref, d2_ref, thr_ref):
    x = x_ref[0]
    y = y_ref[0]
    z = z_ref[0]
    cx = cx_ref[0]
    cy = cy_ref[0]
    cz = cz_ref[0]
    ex = cx - x
    ey = cy - y
    ez = cz - z
    d = ex * ex + ey * ey + ez * ez
    d2_ref[0] = d
    mins = [
        jnp.min(d[:, j * _CW:(j + 1) * _CW], axis=1, keepdims=True)
        for j in range(_NCHUNK)
    ]
    m = jnp.concatenate(mins, axis=1)
    iota_c = lax.broadcasted_iota(jnp.int32, (G, _NCHUNK), 1)
    inf = jnp.float32(jnp.inf)
    for _ in range(K - 1):
        mn = jnp.min(m, axis=1, keepdims=True)
        first = jnp.min(jnp.where(m == mn, iota_c, _NCHUNK), axis=1,
                        keepdims=True)
        m = jnp.where(iota_c == first, inf, m)
    thr_ref[0, 0] = jnp.min(m, axis=1)


def _dist_pallas(xp, yp, zp, cx, cy, cz):
    f32 = jnp.float32
    return pl.pallas_call(
        _dist_body,
        grid=(B,),
        in_specs=[
            pl.BlockSpec((1, 1, N), lambda b: (b, 0, 0)),
            pl.BlockSpec((1, 1, N), lambda b: (b, 0, 0)),
            pl.BlockSpec((1, 1, N), lambda b: (b, 0, 0)),
            pl.BlockSpec((1, G, 1), lambda b: (b, 0, 0)),
            pl.BlockSpec((1, G, 1), lambda b: (b, 0, 0)),
            pl.BlockSpec((1, G, 1), lambda b: (b, 0, 0)),
        ],
        out_specs=[
            pl.BlockSpec((1, G, N), lambda b: (b, 0, 0)),
            pl.BlockSpec((1, 1, G), lambda b: (b, 0, 0)),
        ],
        out_shape=(
            jax.ShapeDtypeStruct((B, G, N), f32),
            jax.ShapeDtypeStruct((B, 1, G), f32),
        ),
    )(xp[:, None, :], yp[:, None, :], zp[:, None, :],
      cx[:, :, None], cy[:, :, None], cz[:, :, None])


_CAND = N + 32
_BIGI = 1 << 30


def _knn_sc_body(d2_h, xp_h, yp_h, zp_h, cx_h, cy_h, cz_h, thr_h,
                 gidx_h, rx_h, ry_h, rz_h,
                 xv, yv, zv, cxv, cyv, czv, tv,
                 dA, dB, cd, ci, oi, ox, oy, oz, sA, sB):
    f32 = jnp.float32
    i32 = jnp.int32
    inf = f32(jnp.inf)
    wid = lax.axis_index("s") * NC + lax.axis_index("c")
    lane = lax.broadcasted_iota(i32, (16,), 0)
    inf_v = jnp.full((16,), inf, f32)
    big_v = jnp.full((16,), _BIGI, i32)

    pltpu.sync_copy(xp_h.at[wid], xv)
    pltpu.sync_copy(yp_h.at[wid], yv)
    pltpu.sync_copy(zp_h.at[wid], zv)
    pltpu.sync_copy(cx_h.at[wid], cxv.at[pl.ds(0, G)])
    pltpu.sync_copy(cy_h.at[wid], cyv.at[pl.ds(0, G)])
    pltpu.sync_copy(cz_h.at[wid], czv.at[pl.ds(0, G)])
    pltpu.sync_copy(thr_h.at[wid], tv.at[pl.ds(0, G)])

    def row_dma(g, buf, sem):
        return pltpu.make_async_copy(d2_h.at[wid, g], buf, sem)

    row_dma(0, dA, sA).start()
    row_dma(1, dB, sB).start()

    def lex_min_lanes(md, mi):
        for sh in (8, 4, 2, 1):
            perm = (lane + sh) & 15
            md2 = jnp.take(md, perm)
            mi2 = jnp.take(mi, perm)
            c = (md2 < md) | ((md2 == md) & (mi2 < mi))
            md = jnp.where(c, md2, md)
            mi = jnp.where(c, mi2, mi)
        return md, mi

    def select_g(g, dref):
        scx = cxv[pl.ds(g, 16)][0]
        scy = cyv[pl.ds(g, 16)][0]
        scz = czv[pl.ds(g, 16)][0]
        t = tv[pl.ds(g, 16)][0]

        def cbody(i, off):
            dv = dref[pl.ds(i * 16, 16)]
            m = dv <= t
            iv = lane + i * 16
            plsc.store_compressed(cd.at[pl.ds(off, 16)], dv, mask=m)
            plsc.store_compressed(ci.at[pl.ds(off, 16)], iv, mask=m)
            pc = plsc.all_reduce_population_count(m)
            return off + pc[0]

        cnt = lax.fori_loop(0, N // 16, cbody, i32(0))
        cd[pl.ds(cnt, 16)] = inf_v
        ci[pl.ds(cnt, 16)] = big_v
        nv = cnt // 16 + 1

        def kbody(k, carry):
            lastd, lasti, acc = carry

            def sbody(j, mm):
                md, mi = mm
                dv = cd[pl.ds(j * 16, 16)]
                iv = ci[pl.ds(j * 16, 16)]
                valid = (dv > lastd) | ((dv == lastd) & (iv > lasti))
                dv2 = jnp.where(valid, dv, inf_v)
                iv2 = jnp.where(valid, iv, big_v)
                better = (dv2 < md) | ((dv2 == md) & (iv2 < mi))
                return (jnp.where(better, dv2, md),
                        jnp.where(better, iv2, mi))

            md, mi = lax.fori_loop(0, nv, sbody, (inf_v, big_v))
            gm, gi = lex_min_lanes(md, mi)
            acc = jnp.where(lane == (k % 16), gi, acc)

            @pl.when(k % 16 == 15)
            def _():
                oi[pl.ds(g * K + (k // 16) * 16, 16)] = acc

            return gm, gi, acc

        lax.fori_loop(0, K, kbody,
                      (jnp.full((16,), -jnp.inf, f32),
                       jnp.full((16,), -1, i32),
                       jnp.zeros((16,), i32)))

        for h in range(K // 16):
            ivv = oi[pl.ds(g * K + h * 16, 16)]
            gx = plsc.load_gather(xv, [ivv])
            gy = plsc.load_gather(yv, [ivv])
            gz = plsc.load_gather(zv, [ivv])
            ox[pl.ds(g * K + h * 16, 16)] = gx - scx
            oy[pl.ds(g * K + h * 16, 16)] = gy - scy
            oz[pl.ds(g * K + h * 16, 16)] = gz - scz

    def pair(i, _):
        g0 = i * 2
        g1 = g0 + 1
        row_dma(g0, dA, sA).wait()
        select_g(g0, dA)

        @pl.when(g0 + 2 < G)
        def _():
            row_dma(g0 + 2, dA, sA).start()

        row_dma(g1, dB, sB).wait()
        select_g(g1, dB)

        @pl.when(g1 + 2 < G)
        def _():
            row_dma(g1 + 2, dB, sB).start()

        return 0

    lax.fori_loop(0, G // 2, pair, 0)

    pltpu.sync_copy(oi, gidx_h.at[wid])
    pltpu.sync_copy(ox, rx_h.at[wid])
    pltpu.sync_copy(oy, ry_h.at[wid])
    pltpu.sync_copy(oz, rz_h.at[wid])


def _knn_sc(d2, xp, yp, zp, cx, cy, cz, thr):
    f32 = jnp.float32
    i32 = jnp.int32
    mesh = plsc.VectorSubcoreMesh(core_axis_name="c", subcore_axis_name="s",
                                  num_cores=NC, num_subcores=NS)
    out_type = (
        jax.ShapeDtypeStruct((B, G * K), i32),
        jax.ShapeDtypeStruct((B, G * K), f32),
        jax.ShapeDtypeStruct((B, G * K), f32),
        jax.ShapeDtypeStruct((B, G * K), f32),
    )
    scratch = [
        pltpu.VMEM((N,), f32),
        pltpu.VMEM((N,), f32),
        pltpu.VMEM((N,), f32),
        pltpu.VMEM((G + 16,), f32),
        pltpu.VMEM((G + 16,), f32),
        pltpu.VMEM((G + 16,), f32),
        pltpu.VMEM((G + 16,), f32),
        pltpu.VMEM((N,), f32),
        pltpu.VMEM((N,), f32),
        pltpu.VMEM((_CAND,), f32),
        pltpu.VMEM((_CAND,), i32),
        pltpu.VMEM((G * K,), i32),
        pltpu.VMEM((G * K,), f32),
        pltpu.VMEM((G * K,), f32),
        pltpu.VMEM((G * K,), f32),
        pltpu.SemaphoreType.DMA,
        pltpu.SemaphoreType.DMA,
    ]
    fn = pl.kernel(_knn_sc_body, out_type=out_type, mesh=mesh,
                   scratch_types=scratch,
                   compiler_params=pltpu.CompilerParams(
                       needs_layout_passes=False))
    return fn(d2, xp, yp, zp, cx, cy, cz, thr)


def kernel(xyz, W1, b1, W2, b2, W3, b3, W4, b4):
    xp = xyz[:, :, 0]
    yp = xyz[:, :, 1]
    zp = xyz[:, :, 2]
    center_idx, cx, cy, cz = _fps_pallas(xp, yp, zp)
    centers_xyz = jnp.stack([cx, cy, cz], axis=-1)

    d2, thr = _dist_pallas(xp, yp, zp, cx, cy, cz)
    gidx, rx, ry, rz = _knn_sc(d2, xp, yp, zp, cx, cy, cz, thr[:, 0, :])
    group_idx = gidx.reshape(B, G, K)
    rel_xyz = jnp.stack([rx, ry, rz], axis=-1)

    h = rel_xyz.reshape(B * G * K, 3)
    h = jax.nn.gelu(h @ W1 + b1, approximate=False)
    h = jax.nn.gelu(h @ W2 + b2, approximate=False)
    h = jax.nn.gelu(h @ W3 + b3, approximate=False)
    h = h @ W4 + b4
    h = h.reshape(B, G, K, EMBED_DIM)
    tokens = h.max(axis=2)
    return tokens, centers_xyz, group_idx


# speedup vs baseline: 1.8944x; 1.8944x over previous
import functools

import jax
import jax.numpy as jnp
from jax import lax
from jax.experimental import pallas as pl
from jax.experimental.pallas import tpu as pltpu
from jax.experimental.pallas import tpu_sc as plsc

B = 32
N = 8192
G = 128
K = 32
EMBED_DIM = 768
HIDDEN = 512

NC = 2
NS = 16


def _fps_body(x_ref, y_ref, z_ref, idx_ref, cx_ref, cy_ref, cz_ref):
    x = x_ref[:]
    y = y_ref[:]
    z = z_ref[:]
    f32 = jnp.float32
    mx = jnp.mean(x, axis=1, keepdims=True)
    my = jnp.mean(y, axis=1, keepdims=True)
    mz = jnp.mean(z, axis=1, keepdims=True)
    dx, dy, dz = x - mx, y - my, z - mz
    dist0 = dx * dx + dy * dy + dz * dz
    iota = lax.broadcasted_iota(jnp.int32, (B, N), 1)

    def argmax_first(d):
        m = jnp.max(d, axis=1, keepdims=True)
        return jnp.min(jnp.where(d == m, iota, N), axis=1, keepdims=True)

    farthest = argmax_first(dist0)
    distance = jnp.full((B, N), 1e10, dtype=f32)
    iota_g = lax.broadcasted_iota(jnp.int32, (B, G), 1)
    acc_i = jnp.zeros((B, G), jnp.int32)
    acc_x = jnp.zeros((B, G), f32)
    acc_y = jnp.zeros((B, G), f32)
    acc_z = jnp.zeros((B, G), f32)

    def step(i, carry):
        distance, farthest, acc_i, acc_x, acc_y, acc_z = carry
        sel = iota == farthest
        zero = jnp.zeros((), f32)
        cx = jnp.sum(jnp.where(sel, x, zero), axis=1, keepdims=True)
        cy = jnp.sum(jnp.where(sel, y, zero), axis=1, keepdims=True)
        cz = jnp.sum(jnp.where(sel, z, zero), axis=1, keepdims=True)
        here = iota_g == i
        acc_i = jnp.where(here, farthest, acc_i)
        acc_x = jnp.where(here, cx, acc_x)
        acc_y = jnp.where(here, cy, acc_y)
        acc_z = jnp.where(here, cz, acc_z)
        ex, ey, ez = x - cx, y - cy, z - cz
        dist = ex * ex + ey * ey + ez * ez
        distance = jnp.minimum(distance, dist)
        farthest = argmax_first(distance)
        return distance, farthest, acc_i, acc_x, acc_y, acc_z

    carry = (distance, farthest, acc_i, acc_x, acc_y, acc_z)
    carry = lax.fori_loop(0, G, step, carry)
    _, _, acc_i, acc_x, acc_y, acc_z = carry
    idx_ref[:] = acc_i
    cx_ref[:] = acc_x
    cy_ref[:] = acc_y
    cz_ref[:] = acc_z


def _fps_pallas(xp, yp, zp):
    out_shapes = (
        jax.ShapeDtypeStruct((B, G), jnp.int32),
        jax.ShapeDtypeStruct((B, G), jnp.float32),
        jax.ShapeDtypeStruct((B, G), jnp.float32),
        jax.ShapeDtypeStruct((B, G), jnp.float32),
    )
    return pl.pallas_call(_fps_body, out_shape=out_shapes)(xp, yp, zp)


_NCHUNK = 64
_CW = N // _NCHUNK


def _dist_body(x_ref, y_ref, z_ref, cx_ref, cy_ref, cz_ref, d2_ref, thr_ref):
    x = x_ref[0]
    y = y_ref[0]
    z = z_ref[0]
    cx = cx_ref[0]
    cy = cy_ref[0]
    cz = cz_ref[0]
    ex = cx - x
    ey = cy - y
    ez = cz - z
    d = ex * ex + ey * ey + ez * ez
    d2_ref[0] = d
    mins = [
        jnp.min(d[:, j * _CW:(j + 1) * _CW], axis=1, keepdims=True)
        for j in range(_NCHUNK)
    ]
    m = jnp.concatenate(mins, axis=1)
    iota_c = lax.broadcasted_iota(jnp.int32, (G, _NCHUNK), 1)
    inf = jnp.float32(jnp.inf)
    for _ in range(K - 1):
        mn = jnp.min(m, axis=1, keepdims=True)
        first = jnp.min(jnp.where(m == mn, iota_c, _NCHUNK), axis=1,
                        keepdims=True)
        m = jnp.where(iota_c == first, inf, m)
    thr_ref[0, 0] = jnp.min(m, axis=1)


def _dist_pallas(xp, yp, zp, cx, cy, cz):
    f32 = jnp.float32
    return pl.pallas_call(
        _dist_body,
        grid=(B,),
        in_specs=[
            pl.BlockSpec((1, 1, N), lambda b: (b, 0, 0)),
            pl.BlockSpec((1, 1, N), lambda b: (b, 0, 0)),
            pl.BlockSpec((1, 1, N), lambda b: (b, 0, 0)),
            pl.BlockSpec((1, G, 1), lambda b: (b, 0, 0)),
            pl.BlockSpec((1, G, 1), lambda b: (b, 0, 0)),
            pl.BlockSpec((1, G, 1), lambda b: (b, 0, 0)),
        ],
        out_specs=[
            pl.BlockSpec((1, G, N), lambda b: (b, 0, 0)),
            pl.BlockSpec((1, 1, G), lambda b: (b, 0, 0)),
        ],
        out_shape=(
            jax.ShapeDtypeStruct((B, G, N), f32),
            jax.ShapeDtypeStruct((B, 1, G), f32),
        ),
    )(xp[:, None, :], yp[:, None, :], zp[:, None, :],
      cx[:, :, None], cy[:, :, None], cz[:, :, None])


_CAND = N + 32
_BIGI = 1 << 30


def _knn_sc_body(d2_h, xp_h, yp_h, zp_h, cx_h, cy_h, cz_h, thr_h,
                 gidx_h, rx_h, ry_h, rz_h,
                 xv, yv, zv, cxv, cyv, czv, tv,
                 dA, dB, cd, ci, oi, ox, oy, oz, sA, sB):
    f32 = jnp.float32
    i32 = jnp.int32
    inf = f32(jnp.inf)
    wid = lax.axis_index("s") * NC + lax.axis_index("c")
    lane = lax.broadcasted_iota(i32, (16,), 0)
    inf_v = jnp.full((16,), inf, f32)
    big_v = jnp.full((16,), _BIGI, i32)

    pltpu.sync_copy(xp_h.at[wid], xv)
    pltpu.sync_copy(yp_h.at[wid], yv)
    pltpu.sync_copy(zp_h.at[wid], zv)
    pltpu.sync_copy(cx_h.at[wid], cxv.at[pl.ds(0, G)])
    pltpu.sync_copy(cy_h.at[wid], cyv.at[pl.ds(0, G)])
    pltpu.sync_copy(cz_h.at[wid], czv.at[pl.ds(0, G)])
    pltpu.sync_copy(thr_h.at[wid], tv.at[pl.ds(0, G)])

    def row_dma(g, buf, sem):
        return pltpu.make_async_copy(d2_h.at[wid, g], buf, sem)

    row_dma(0, dA, sA).start()
    row_dma(1, dB, sB).start()

    def lex_min_lanes(md, mi):
        for sh in (8, 4, 2, 1):
            perm = (lane + sh) & 15
            md2 = jnp.take(md, perm)
            mi2 = jnp.take(mi, perm)
            c = (md2 < md) | ((md2 == md) & (mi2 < mi))
            md = jnp.where(c, md2, md)
            mi = jnp.where(c, mi2, mi)
        return md, mi

    def select_g(g, dref):
        scx = cxv[pl.ds(g, 16)][0]
        scy = cyv[pl.ds(g, 16)][0]
        scz = czv[pl.ds(g, 16)][0]
        t = tv[pl.ds(g, 16)][0]

        def cbody(i, off):
            dv = dref[pl.ds(i * 16, 16)]
            m = dv <= t
            iv = lane + i * 16
            plsc.store_compressed(cd.at[pl.ds(off, 16)], dv, mask=m)
            plsc.store_compressed(ci.at[pl.ds(off, 16)], iv, mask=m)
            pc = plsc.all_reduce_population_count(m)
            return off + pc[0]

        cnt = lax.fori_loop(0, N // 16, cbody, i32(0))
        cd[pl.ds(cnt, 16)] = inf_v
        ci[pl.ds(cnt, 16)] = big_v
        nv = cnt // 16 + 1

        def kbody(k, carry):
            lastd, lasti, acc = carry

            def sbody(j, mm):
                md, mi = mm
                dv = cd[pl.ds(j * 16, 16)]
                iv = ci[pl.ds(j * 16, 16)]
                valid = (dv > lastd) | ((dv == lastd) & (iv > lasti))
                dv2 = jnp.where(valid, dv, inf_v)
                iv2 = jnp.where(valid, iv, big_v)
                better = (dv2 < md) | ((dv2 == md) & (iv2 < mi))
                return (jnp.where(better, dv2, md),
                        jnp.where(better, iv2, mi))

            md, mi = lax.fori_loop(0, nv, sbody, (inf_v, big_v))
            gm, gi = lex_min_lanes(md, mi)
            acc = jnp.where(lane == (k % 16), gi, acc)

            @pl.when(k % 16 == 15)
            def _():
                oi[pl.ds(g * K + (k // 16) * 16, 16)] = acc

            return gm, gi, acc

        lax.fori_loop(0, K, kbody,
                      (jnp.full((16,), -jnp.inf, f32),
                       jnp.full((16,), -1, i32),
                       jnp.zeros((16,), i32)))

        for h in range(K // 16):
            ivv = oi[pl.ds(g * K + h * 16, 16)]
            gx = plsc.load_gather(xv, [ivv])
            gy = plsc.load_gather(yv, [ivv])
            gz = plsc.load_gather(zv, [ivv])
            ox[pl.ds(g * K + h * 16, 16)] = gx - scx
            oy[pl.ds(g * K + h * 16, 16)] = gy - scy
            oz[pl.ds(g * K + h * 16, 16)] = gz - scz

    def pair(i, _):
        g0 = i * 2
        g1 = g0 + 1
        row_dma(g0, dA, sA).wait()
        select_g(g0, dA)

        @pl.when(g0 + 2 < G)
        def _():
            row_dma(g0 + 2, dA, sA).start()

        row_dma(g1, dB, sB).wait()
        select_g(g1, dB)

        @pl.when(g1 + 2 < G)
        def _():
            row_dma(g1 + 2, dB, sB).start()

        return 0

    lax.fori_loop(0, G // 2, pair, 0)

    pltpu.sync_copy(oi, gidx_h.at[wid])
    pltpu.sync_copy(ox, rx_h.at[wid])
    pltpu.sync_copy(oy, ry_h.at[wid])
    pltpu.sync_copy(oz, rz_h.at[wid])


def _knn_sc(d2, xp, yp, zp, cx, cy, cz, thr):
    f32 = jnp.float32
    i32 = jnp.int32
    mesh = plsc.VectorSubcoreMesh(core_axis_name="c", subcore_axis_name="s",
                                  num_cores=NC, num_subcores=NS)
    out_type = (
        jax.ShapeDtypeStruct((B, G * K), i32),
        jax.ShapeDtypeStruct((B, G * K), f32),
        jax.ShapeDtypeStruct((B, G * K), f32),
        jax.ShapeDtypeStruct((B, G * K), f32),
    )
    scratch = [
        pltpu.VMEM((N,), f32),
        pltpu.VMEM((N,), f32),
        pltpu.VMEM((N,), f32),
        pltpu.VMEM((G + 16,), f32),
        pltpu.VMEM((G + 16,), f32),
        pltpu.VMEM((G + 16,), f32),
        pltpu.VMEM((G + 16,), f32),
        pltpu.VMEM((N,), f32),
        pltpu.VMEM((N,), f32),
        pltpu.VMEM((_CAND,), f32),
        pltpu.VMEM((_CAND,), i32),
        pltpu.VMEM((G * K,), i32),
        pltpu.VMEM((G * K,), f32),
        pltpu.VMEM((G * K,), f32),
        pltpu.VMEM((G * K,), f32),
        pltpu.SemaphoreType.DMA,
        pltpu.SemaphoreType.DMA,
    ]
    fn = pl.kernel(_knn_sc_body, out_type=out_type, mesh=mesh,
                   scratch_types=scratch,
                   compiler_params=pltpu.CompilerParams(
                       needs_layout_passes=False))
    return fn(d2, xp, yp, zp, cx, cy, cz, thr)


_BT = 2048
_M = B * G * K


def _mlp_body(rx_ref, ry_ref, rz_ref, w1_ref, b1_ref, w2_ref, b2_ref,
              w3_ref, b3_ref, w4_ref, b4_ref, out_ref):
    f32 = jnp.float32
    bf16 = jnp.bfloat16

    def gelu(v):
        return 0.5 * v * (1.0 + lax.erf(v * 0.7071067811865476))

    x3 = jnp.concatenate([rx_ref[:], ry_ref[:], rz_ref[:]], axis=1)
    h = lax.dot_general(x3, w1_ref[:], (((1,), (0,)), ((), ())),
                        preferred_element_type=f32) + b1_ref[:]
    h = gelu(h)
    h = lax.dot_general(h.astype(bf16), w2_ref[:].astype(bf16),
                        (((1,), (0,)), ((), ())),
                        preferred_element_type=f32) + b2_ref[:]
    h = gelu(h)
    h = lax.dot_general(h.astype(bf16), w3_ref[:].astype(bf16),
                        (((1,), (0,)), ((), ())),
                        preferred_element_type=f32) + b3_ref[:]
    h = gelu(h)
    h = lax.dot_general(h.astype(bf16), w4_ref[:].astype(bf16),
                        (((1,), (0,)), ((), ())),
                        preferred_element_type=f32) + b4_ref[:]
    out_ref[:] = jnp.max(h.reshape(_BT // K, K, EMBED_DIM), axis=1)


def _mlp_pallas(rx, ry, rz, W1, b1, W2, b2, W3, b3, W4, b4):
    f32 = jnp.float32
    steps = _M // _BT
    col = pl.BlockSpec((_BT, 1), lambda i: (i, 0))
    full = lambda a: pl.BlockSpec(a.shape, lambda i: (0,) * a.ndim)
    return pl.pallas_call(
        _mlp_body,
        grid=(steps,),
        in_specs=[col, col, col,
                  full(W1), full(b1[None]), full(W2), full(b2[None]),
                  full(W3), full(b3[None]), full(W4), full(b4[None])],
        out_specs=pl.BlockSpec((_BT // K, EMBED_DIM), lambda i: (i, 0)),
        out_shape=jax.ShapeDtypeStruct((_M // K, EMBED_DIM), f32),
    )(rx.reshape(_M, 1), ry.reshape(_M, 1), rz.reshape(_M, 1),
      W1, b1[None], W2, b2[None], W3, b3[None], W4, b4[None])


def kernel(xyz, W1, b1, W2, b2, W3, b3, W4, b4):
    xp = xyz[:, :, 0]
    yp = xyz[:, :, 1]
    zp = xyz[:, :, 2]
    center_idx, cx, cy, cz = _fps_pallas(xp, yp, zp)
    centers_xyz = jnp.stack([cx, cy, cz], axis=-1)

    d2, thr = _dist_pallas(xp, yp, zp, cx, cy, cz)
    gidx, rx, ry, rz = _knn_sc(d2, xp, yp, zp, cx, cy, cz, thr[:, 0, :])
    group_idx = gidx.reshape(B, G, K)
    tokens = _mlp_pallas(rx, ry, rz, W1, b1, W2, b2, W3, b3, W4, b4)
    tokens = tokens.reshape(B, G, EMBED_DIM)
    return tokens, centers_xyz, group_idx


# speedup vs baseline: 2.8758x; 1.5181x over previous
import functools

import jax
import jax.numpy as jnp
from jax import lax
from jax.experimental import pallas as pl
from jax.experimental.pallas import tpu as pltpu
from jax.experimental.pallas import tpu_sc as plsc

B = 32
N = 8192
G = 128
K = 32
EMBED_DIM = 768
HIDDEN = 512

NC = 2
NS = 16


def _fps_body(x_ref, y_ref, z_ref, idx_ref, cx_ref, cy_ref, cz_ref):
    x = x_ref[:]
    y = y_ref[:]
    z = z_ref[:]
    f32 = jnp.float32
    mx = jnp.mean(x, axis=1, keepdims=True)
    my = jnp.mean(y, axis=1, keepdims=True)
    mz = jnp.mean(z, axis=1, keepdims=True)
    dx, dy, dz = x - mx, y - my, z - mz
    dist0 = dx * dx + dy * dy + dz * dz
    iota = lax.broadcasted_iota(jnp.int32, (B, N), 1)

    def argmax_first(d):
        m = jnp.max(d, axis=1, keepdims=True)
        return jnp.min(jnp.where(d == m, iota, N), axis=1, keepdims=True)

    farthest = argmax_first(dist0)
    distance = jnp.full((B, N), 1e10, dtype=f32)
    iota_g = lax.broadcasted_iota(jnp.int32, (B, G), 1)
    acc_i = jnp.zeros((B, G), jnp.int32)
    acc_x = jnp.zeros((B, G), f32)
    acc_y = jnp.zeros((B, G), f32)
    acc_z = jnp.zeros((B, G), f32)

    def step(i, carry):
        distance, farthest, acc_i, acc_x, acc_y, acc_z = carry
        sel = iota == farthest
        zero = jnp.zeros((), f32)
        cx = jnp.sum(jnp.where(sel, x, zero), axis=1, keepdims=True)
        cy = jnp.sum(jnp.where(sel, y, zero), axis=1, keepdims=True)
        cz = jnp.sum(jnp.where(sel, z, zero), axis=1, keepdims=True)
        here = iota_g == i
        acc_i = jnp.where(here, farthest, acc_i)
        acc_x = jnp.where(here, cx, acc_x)
        acc_y = jnp.where(here, cy, acc_y)
        acc_z = jnp.where(here, cz, acc_z)
        ex, ey, ez = x - cx, y - cy, z - cz
        dist = ex * ex + ey * ey + ez * ez
        distance = jnp.minimum(distance, dist)
        farthest = argmax_first(distance)
        return distance, farthest, acc_i, acc_x, acc_y, acc_z

    carry = (distance, farthest, acc_i, acc_x, acc_y, acc_z)
    carry = lax.fori_loop(0, G, step, carry)
    _, _, acc_i, acc_x, acc_y, acc_z = carry
    idx_ref[:] = acc_i
    cx_ref[:] = acc_x
    cy_ref[:] = acc_y
    cz_ref[:] = acc_z


def _fps_pallas(xp, yp, zp):
    out_shapes = (
        jax.ShapeDtypeStruct((B, G), jnp.int32),
        jax.ShapeDtypeStruct((B, G), jnp.float32),
        jax.ShapeDtypeStruct((B, G), jnp.float32),
        jax.ShapeDtypeStruct((B, G), jnp.float32),
    )
    return pl.pallas_call(_fps_body, out_shape=out_shapes)(xp, yp, zp)


_NCHUNK = 64
_CW = N // _NCHUNK


def _dist_body(x_ref, y_ref, z_ref, cx_ref, cy_ref, cz_ref, d2_ref, thr_ref):
    x = x_ref[0]
    y = y_ref[0]
    z = z_ref[0]
    cx = cx_ref[0]
    cy = cy_ref[0]
    cz = cz_ref[0]
    ex = cx - x
    ey = cy - y
    ez = cz - z
    d = ex * ex + ey * ey + ez * ez
    d2_ref[0] = d
    mins = [
        jnp.min(d[:, j * _CW:(j + 1) * _CW], axis=1, keepdims=True)
        for j in range(_NCHUNK)
    ]
    m = jnp.concatenate(mins, axis=1)
    iota_c = lax.broadcasted_iota(jnp.int32, (G, _NCHUNK), 1)
    inf = jnp.float32(jnp.inf)
    for _ in range(K - 1):
        mn = jnp.min(m, axis=1, keepdims=True)
        first = jnp.min(jnp.where(m == mn, iota_c, _NCHUNK), axis=1,
                        keepdims=True)
        m = jnp.where(iota_c == first, inf, m)
    thr_ref[0, 0] = jnp.min(m, axis=1)


def _dist_pallas(xp, yp, zp, cx, cy, cz):
    f32 = jnp.float32
    return pl.pallas_call(
        _dist_body,
        grid=(B,),
        in_specs=[
            pl.BlockSpec((1, 1, N), lambda b: (b, 0, 0)),
            pl.BlockSpec((1, 1, N), lambda b: (b, 0, 0)),
            pl.BlockSpec((1, 1, N), lambda b: (b, 0, 0)),
            pl.BlockSpec((1, G, 1), lambda b: (b, 0, 0)),
            pl.BlockSpec((1, G, 1), lambda b: (b, 0, 0)),
            pl.BlockSpec((1, G, 1), lambda b: (b, 0, 0)),
        ],
        out_specs=[
            pl.BlockSpec((1, G, N), lambda b: (b, 0, 0)),
            pl.BlockSpec((1, 1, G), lambda b: (b, 0, 0)),
        ],
        out_shape=(
            jax.ShapeDtypeStruct((B, G, N), f32),
            jax.ShapeDtypeStruct((B, 1, G), f32),
        ),
    )(xp[:, None, :], yp[:, None, :], zp[:, None, :],
      cx[:, :, None], cy[:, :, None], cz[:, :, None])


_CAND = N + 32
_BIGI = 1 << 30


def _knn_sc_body(d2_h, xp_h, yp_h, zp_h, cx_h, cy_h, cz_h, thr_h,
                 gidx_h, rx_h, ry_h, rz_h,
                 xv, yv, zv, cxv, cyv, czv, tv,
                 dA, dB, cd, ci, oi, ox, oy, oz, sA, sB):
    f32 = jnp.float32
    i32 = jnp.int32
    inf = f32(jnp.inf)
    wid = lax.axis_index("s") * NC + lax.axis_index("c")
    lane = lax.broadcasted_iota(i32, (16,), 0)
    inf_v = jnp.full((16,), inf, f32)
    big_v = jnp.full((16,), _BIGI, i32)

    pltpu.sync_copy(xp_h.at[wid], xv)
    pltpu.sync_copy(yp_h.at[wid], yv)
    pltpu.sync_copy(zp_h.at[wid], zv)
    pltpu.sync_copy(cx_h.at[wid], cxv.at[pl.ds(0, G)])
    pltpu.sync_copy(cy_h.at[wid], cyv.at[pl.ds(0, G)])
    pltpu.sync_copy(cz_h.at[wid], czv.at[pl.ds(0, G)])
    pltpu.sync_copy(thr_h.at[wid], tv.at[pl.ds(0, G)])

    def row_dma(g, buf, sem):
        return pltpu.make_async_copy(d2_h.at[wid, g], buf, sem)

    row_dma(0, dA, sA).start()
    row_dma(1, dB, sB).start()

    def lex_min_lanes(md, mi):
        for sh in (8, 4, 2, 1):
            perm = (lane + sh) & 15
            md2 = jnp.take(md, perm)
            mi2 = jnp.take(mi, perm)
            c = (md2 < md) | ((md2 == md) & (mi2 < mi))
            md = jnp.where(c, md2, md)
            mi = jnp.where(c, mi2, mi)
        return md, mi

    def select_g(g, dref):
        scx = cxv[pl.ds(g, 16)][0]
        scy = cyv[pl.ds(g, 16)][0]
        scz = czv[pl.ds(g, 16)][0]
        t = tv[pl.ds(g, 16)][0]

        @plsc.parallel_loop(0, N // 16, carry=i32(0), unroll=8)
        def cnt(i, off):
            dv = dref[pl.ds(i * 16, 16)]
            m = dv <= t
            iv = lane + i * 16
            plsc.store_compressed(cd.at[pl.ds(off, 16)], dv, mask=m)
            plsc.store_compressed(ci.at[pl.ds(off, 16)], iv, mask=m)
            pc = plsc.all_reduce_population_count(m)
            return off + pc[0]
        cd[pl.ds(cnt, 16)] = inf_v
        ci[pl.ds(cnt, 16)] = big_v
        nv = cnt // 16 + 1

        def kbody(k, carry):
            lastd, lasti, acc = carry

            def sbody(j, mm):
                md, mi = mm
                dv = cd[pl.ds(j * 16, 16)]
                iv = ci[pl.ds(j * 16, 16)]
                valid = (dv > lastd) | ((dv == lastd) & (iv > lasti))
                dv2 = jnp.where(valid, dv, inf_v)
                iv2 = jnp.where(valid, iv, big_v)
                better = (dv2 < md) | ((dv2 == md) & (iv2 < mi))
                return (jnp.where(better, dv2, md),
                        jnp.where(better, iv2, mi))

            md, mi = lax.fori_loop(0, nv, sbody, (inf_v, big_v))
            gm, gi = lex_min_lanes(md, mi)
            acc = jnp.where(lane == (k % 16), gi, acc)

            @pl.when(k % 16 == 15)
            def _():
                oi[pl.ds(g * K + (k // 16) * 16, 16)] = acc

            return gm, gi, acc

        lax.fori_loop(0, K, kbody,
                      (jnp.full((16,), -jnp.inf, f32),
                       jnp.full((16,), -1, i32),
                       jnp.zeros((16,), i32)))

        for h in range(K // 16):
            ivv = oi[pl.ds(g * K + h * 16, 16)]
            gx = plsc.load_gather(xv, [ivv])
            gy = plsc.load_gather(yv, [ivv])
            gz = plsc.load_gather(zv, [ivv])
            ox[pl.ds(g * K + h * 16, 16)] = gx - scx
            oy[pl.ds(g * K + h * 16, 16)] = gy - scy
            oz[pl.ds(g * K + h * 16, 16)] = gz - scz

    def pair(i, _):
        g0 = i * 2
        g1 = g0 + 1
        row_dma(g0, dA, sA).wait()
        select_g(g0, dA)

        @pl.when(g0 + 2 < G)
        def _():
            row_dma(g0 + 2, dA, sA).start()

        row_dma(g1, dB, sB).wait()
        select_g(g1, dB)

        @pl.when(g1 + 2 < G)
        def _():
            row_dma(g1 + 2, dB, sB).start()

        return 0

    lax.fori_loop(0, G // 2, pair, 0)

    pltpu.sync_copy(oi, gidx_h.at[wid])
    pltpu.sync_copy(ox, rx_h.at[wid])
    pltpu.sync_copy(oy, ry_h.at[wid])
    pltpu.sync_copy(oz, rz_h.at[wid])


def _knn_sc(d2, xp, yp, zp, cx, cy, cz, thr):
    f32 = jnp.float32
    i32 = jnp.int32
    mesh = plsc.VectorSubcoreMesh(core_axis_name="c", subcore_axis_name="s",
                                  num_cores=NC, num_subcores=NS)
    out_type = (
        jax.ShapeDtypeStruct((B, G * K), i32),
        jax.ShapeDtypeStruct((B, G * K), f32),
        jax.ShapeDtypeStruct((B, G * K), f32),
        jax.ShapeDtypeStruct((B, G * K), f32),
    )
    scratch = [
        pltpu.VMEM((N,), f32),
        pltpu.VMEM((N,), f32),
        pltpu.VMEM((N,), f32),
        pltpu.VMEM((G + 16,), f32),
        pltpu.VMEM((G + 16,), f32),
        pltpu.VMEM((G + 16,), f32),
        pltpu.VMEM((G + 16,), f32),
        pltpu.VMEM((N,), f32),
        pltpu.VMEM((N,), f32),
        pltpu.VMEM((_CAND,), f32),
        pltpu.VMEM((_CAND,), i32),
        pltpu.VMEM((G * K,), i32),
        pltpu.VMEM((G * K,), f32),
        pltpu.VMEM((G * K,), f32),
        pltpu.VMEM((G * K,), f32),
        pltpu.SemaphoreType.DMA,
        pltpu.SemaphoreType.DMA,
    ]
    fn = pl.kernel(_knn_sc_body, out_type=out_type, mesh=mesh,
                   scratch_types=scratch,
                   compiler_params=pltpu.CompilerParams(
                       needs_layout_passes=False))
    return fn(d2, xp, yp, zp, cx, cy, cz, thr)


_BT = 2048
_M = B * G * K


def _mlp_body(rx_ref, ry_ref, rz_ref, w1_ref, b1_ref, w2_ref, b2_ref,
              w3_ref, b3_ref, w4_ref, b4_ref, out_ref):
    f32 = jnp.float32
    bf16 = jnp.bfloat16

    def gelu(v):
        return 0.5 * v * (1.0 + lax.erf(v * 0.7071067811865476))

    x3 = jnp.concatenate([rx_ref[:], ry_ref[:], rz_ref[:]], axis=1)
    h = lax.dot_general(x3, w1_ref[:], (((1,), (0,)), ((), ())),
                        preferred_element_type=f32) + b1_ref[:]
    h = gelu(h)
    h = lax.dot_general(h.astype(bf16), w2_ref[:].astype(bf16),
                        (((1,), (0,)), ((), ())),
                        preferred_element_type=f32) + b2_ref[:]
    h = gelu(h)
    h = lax.dot_general(h.astype(bf16), w3_ref[:].astype(bf16),
                        (((1,), (0,)), ((), ())),
                        preferred_element_type=f32) + b3_ref[:]
    h = gelu(h)
    h = lax.dot_general(h.astype(bf16), w4_ref[:].astype(bf16),
                        (((1,), (0,)), ((), ())),
                        preferred_element_type=f32) + b4_ref[:]
    out_ref[:] = jnp.max(h.reshape(_BT // K, K, EMBED_DIM), axis=1)


def _mlp_pallas(rx, ry, rz, W1, b1, W2, b2, W3, b3, W4, b4):
    f32 = jnp.float32
    steps = _M // _BT
    col = pl.BlockSpec((_BT, 1), lambda i: (i, 0))
    full = lambda a: pl.BlockSpec(a.shape, lambda i: (0,) * a.ndim)
    return pl.pallas_call(
        _mlp_body,
        grid=(steps,),
        in_specs=[col, col, col,
                  full(W1), full(b1[None]), full(W2), full(b2[None]),
                  full(W3), full(b3[None]), full(W4), full(b4[None])],
        out_specs=pl.BlockSpec((_BT // K, EMBED_DIM), lambda i: (i, 0)),
        out_shape=jax.ShapeDtypeStruct((_M // K, EMBED_DIM), f32),
    )(rx.reshape(_M, 1), ry.reshape(_M, 1), rz.reshape(_M, 1),
      W1, b1[None], W2, b2[None], W3, b3[None], W4, b4[None])


def kernel(xyz, W1, b1, W2, b2, W3, b3, W4, b4):
    xp = xyz[:, :, 0]
    yp = xyz[:, :, 1]
    zp = xyz[:, :, 2]
    center_idx, cx, cy, cz = _fps_pallas(xp, yp, zp)
    centers_xyz = jnp.stack([cx, cy, cz], axis=-1)

    d2, thr = _dist_pallas(xp, yp, zp, cx, cy, cz)
    gidx, rx, ry, rz = _knn_sc(d2, xp, yp, zp, cx, cy, cz, thr[:, 0, :])
    group_idx = gidx.reshape(B, G, K)
    tokens = _mlp_pallas(rx, ry, rz, W1, b1, W2, b2, W3, b3, W4, b4)
    tokens = tokens.reshape(B, G, EMBED_DIM)
    return tokens, centers_xyz, group_idx
